# Initial kernel scaffold; baseline (speedup 1.0000x reference)
import functools

import jax
import jax.numpy as jnp
from jax import lax
from jax.experimental import pallas as pl
from jax.experimental.pallas import tpu as pltpu
from jax.experimental.pallas import tpu_sc as plsc

N = 10000
E = 5000
D = 128
NNZ = 320000

NC, NS, L = 2, 16, 16
NW = NC * NS

NP = 10240
EP = 5120
C = 128
RT = 79
NNZP = NW * RT * C

PER_T = NNZP // NS
HIST_IT = PER_T // L

_mesh = plsc.VectorSubcoreMesh(core_axis_name="c", subcore_axis_name="s")

_f32 = jnp.float32
_i32 = jnp.int32


def _zero16():
    return jnp.zeros((L,), _f32)


def _splat(ref2d, b, j):
    rb = jnp.full((L,), b, _i32)
    cj = jnp.full((L,), j, _i32)
    return plsc.load_gather(ref2d, [rb, cj])


@functools.partial(
    pl.kernel,
    mesh=_mesh,
    out_type=(
        jax.ShapeDtypeStruct((EP // L, L), _f32),
        jax.ShapeDtypeStruct((NP, D), _f32),
    ),
    scratch_types=[
        pltpu.VMEM((PER_T,), _i32),
        pltpu.VMEM((NP // L, L), _f32),
        pltpu.VMEM((EP // L, L), _f32),
        pltpu.VMEM((10, 64), _i32),
        pltpu.VMEM((40, L), _f32),
        pltpu.VMEM((20, L), _f32),
        pltpu.VMEM((320, D), _f32),
        pltpu.VMEM_SHARED((NP // L, L), _f32),
        pltpu.VMEM_SHARED((EP // L, L), _f32),
    ],
)
def _stage_a(xpad, nidx, eidx, inv_e_out, xnorm_out,
             idxbuf, histv, histe, rid, invbuf, sinv, xbuf, accv, acce):
    c = lax.axis_index("c")
    s = lax.axis_index("s")
    w = s * NC + c
    z16 = _zero16()
    ones = jnp.ones((L,), _f32)
    iota = lax.iota(_i32, L)

    def _zv(r, _):
        histv[r] = z16
        return 0
    lax.fori_loop(0, NP // L, _zv, 0)

    def _ze(r, _):
        histe[r] = z16
        return 0
    lax.fori_loop(0, EP // L, _ze, 0)

    for j in range(10):
        for k in range(4):
            rid[j, pl.ds(k * L, L)] = iota + (j * 64 + k * L)

    pltpu.sync_copy(nidx.at[pl.ds(s * PER_T, PER_T)], idxbuf)

    def _hv(i, _):
        v = idxbuf[pl.ds(i * L, L)]
        plsc.addupdate_scatter(histv, [v >> 4, v & 15], ones)
        return 0
    lax.fori_loop(0, HIST_IT, _hv, 0)

    pltpu.sync_copy(eidx.at[pl.ds(s * PER_T, PER_T)], idxbuf)

    def _he(i, _):
        v = idxbuf[pl.ds(i * L, L)]
        plsc.addupdate_scatter(histe, [v >> 4, v & 15], ones)
        return 0
    lax.fori_loop(0, HIST_IT, _he, 0)

    def _zi(r, _):
        invbuf[r] = z16
        return 0
    lax.fori_loop(0, 40, _zi, 0)
    pltpu.sync_copy(invbuf, accv.at[pl.ds(s * 40, 40)])
    pltpu.sync_copy(invbuf.at[pl.ds(0, 20)], acce.at[pl.ds(s * 20, 20)])
    plsc.subcore_barrier()

    for j in range(10):
        pltpu.sync_copy(histv.at[pl.ds(j * 64, 64)], accv.at[rid.at[j]],
                        add=True)
    for j in range(5):
        pltpu.sync_copy(histe.at[pl.ds(j * 64, 64)], acce.at[rid.at[j]],
                        add=True)
    plsc.subcore_barrier()

    pltpu.sync_copy(accv.at[pl.ds(s * 40, 40)], invbuf)

    def _iv(r, _):
        d = invbuf[r]
        invbuf[r] = jnp.where(d > 0.0, 1.0 / d, 0.0)
        return 0
    lax.fori_loop(0, 40, _iv, 0)
    pltpu.sync_copy(invbuf, accv.at[pl.ds(s * 40, 40)])

    pltpu.sync_copy(acce.at[pl.ds(s * 20, 20)], invbuf.at[pl.ds(0, 20)])

    def _ie(r, _):
        d = invbuf[r]
        invbuf[r] = jnp.where(d > 0.0, 1.0 / d, 0.0)
        return 0
    lax.fori_loop(0, 20, _ie, 0)

    @pl.when(c == 0)
    def _():
        pltpu.sync_copy(invbuf.at[pl.ds(0, 20)],
                        inv_e_out.at[pl.ds(s * 20, 20)])
    plsc.subcore_barrier()

    pltpu.sync_copy(xpad.at[pl.ds(w * 320, 320)], xbuf)
    pltpu.sync_copy(accv.at[pl.ds(w * 20, 20)], sinv)

    def _sc(b, _):
        for j in range(L):
            scale = _splat(sinv, b, j)
            r = b * L + j
            for g in range(D // L):
                xbuf[r, pl.ds(g * L, L)] = xbuf[r, pl.ds(g * L, L)] * scale
        return 0
    lax.fori_loop(0, 20, _sc, 0)
    pltpu.sync_copy(xbuf, xnorm_out.at[pl.ds(w * 320, 320)])


def _make_agg(rows_out):
    @functools.partial(
        pl.kernel,
        mesh=_mesh,
        out_type=jax.ShapeDtypeStruct((NC, rows_out, D), _f32),
        scratch_types=[
            pltpu.VMEM((RT, C), _i32),
            pltpu.VMEM((RT, C), _i32),
            pltpu.VMEM((C, D), _f32),
            pltpu.VMEM((64, D), _f32),
            pltpu.VMEM_SHARED((rows_out, D), _f32),
            pltpu.SemaphoreType.DMA,
        ],
    )
    def _agg(table, gidx, sidx, part_out, gbuf, sbuf, rows, zbuf, acc, sem):
        c = lax.axis_index("c")
        s = lax.axis_index("s")
        w = s * NC + c
        per_t = rows_out // NS
        z16 = _zero16()

        pltpu.sync_copy(gidx.at[pl.ds(w * RT, RT)], gbuf)
        pltpu.sync_copy(sidx.at[pl.ds(w * RT, RT)], sbuf)

        def _z(r, _):
            for g in range(D // L):
                zbuf[r, pl.ds(g * L, L)] = z16
            return 0
        lax.fori_loop(0, 64, _z, 0)
        for k in range(per_t // 64):
            pltpu.sync_copy(zbuf, acc.at[pl.ds(s * per_t + k * 64, 64)])
        plsc.subcore_barrier()

        def _body(j, _):
            pltpu.async_copy(table.at[gbuf.at[j]], rows, sem).wait()
            pltpu.sync_copy(rows, acc.at[sbuf.at[j]], add=True)
            return 0
        lax.fori_loop(0, RT, _body, 0)
        plsc.subcore_barrier()

        for k in range(per_t // 64):
            o = s * per_t + k * 64
            pltpu.sync_copy(acc.at[pl.ds(o, 64)], part_out.at[c, pl.ds(o, 64)])

    return _agg


_v2e = _make_agg(EP)
_e2v = _make_agg(NP)


@functools.partial(
    pl.kernel,
    mesh=_mesh,
    out_type=(
        jax.ShapeDtypeStruct((EP, D), _f32),
        jax.ShapeDtypeStruct((EP, D), _f32),
    ),
    scratch_types=[
        pltpu.VMEM((160, D), _f32),
        pltpu.VMEM((160, D), _f32),
        pltpu.VMEM((160, D), _f32),
        pltpu.VMEM((160, D), _f32),
        pltpu.VMEM((10, L), _f32),
    ],
)
def _stage_c(part, inv_e, efeat_out, enorm_out, p0, p1, eb, nb, sinv):
    c = lax.axis_index("c")
    s = lax.axis_index("s")
    w = s * NC + c
    pltpu.sync_copy(part.at[0, pl.ds(w * 160, 160)], p0)
    pltpu.sync_copy(part.at[1, pl.ds(w * 160, 160)], p1)
    pltpu.sync_copy(inv_e.at[pl.ds(w * 10, 10)], sinv)

    def _cb(b, _):
        for j in range(L):
            scale = _splat(sinv, b, j)
            r = b * L + j
            for g in range(D // L):
                v = p0[r, pl.ds(g * L, L)] + p1[r, pl.ds(g * L, L)]
                eb[r, pl.ds(g * L, L)] = v
                nb[r, pl.ds(g * L, L)] = v * scale
        return 0
    lax.fori_loop(0, 10, _cb, 0)
    pltpu.sync_copy(eb, efeat_out.at[pl.ds(w * 160, 160)])
    pltpu.sync_copy(nb, enorm_out.at[pl.ds(w * 160, 160)])


@functools.partial(
    pl.kernel,
    mesh=_mesh,
    out_type=jax.ShapeDtypeStruct((NP, D), _f32),
    scratch_types=[
        pltpu.VMEM((160, D), _f32),
        pltpu.VMEM((160, D), _f32),
        pltpu.VMEM((160, D), _f32),
    ],
)
def _stage_e(part, nfeat_out, p0, p1, ob):
    c = lax.axis_index("c")
    s = lax.axis_index("s")
    w = s * NC + c
    for h in range(2):
        base = w * 320 + h * 160
        pltpu.sync_copy(part.at[0, pl.ds(base, 160)], p0)
        pltpu.sync_copy(part.at[1, pl.ds(base, 160)], p1)

        def _ad(r, _):
            for g in range(D // L):
                ob[r, pl.ds(g * L, L)] = (p0[r, pl.ds(g * L, L)]
                                          + p1[r, pl.ds(g * L, L)])
            return 0
        lax.fori_loop(0, 160, _ad, 0)
        pltpu.sync_copy(ob, nfeat_out.at[pl.ds(base, 160)])


def kernel(X, node_idx, edge_idx):
    X = X.astype(_f32)
    nidx = node_idx.astype(_i32)
    eidx = edge_idx.astype(_i32)

    xpad = jnp.zeros((NP, D), _f32).at[:N].set(X)
    nidx_p = jnp.full((NNZP,), NP - 1, _i32).at[:NNZ].set(nidx)
    eidx_p = jnp.full((NNZP,), EP - 1, _i32).at[:NNZ].set(eidx)
    nidx2 = nidx_p.reshape(NNZP // C, C)
    eidx2 = eidx_p.reshape(NNZP // C, C)

    inv_e, xnorm = _stage_a(xpad, nidx_p, eidx_p)
    epart = _v2e(xnorm, nidx2, eidx2)
    efeat, enorm = _stage_c(epart, inv_e)
    npart = _e2v(enorm, eidx2, nidx2)
    nfeat = _stage_e(npart)
    return (nfeat[:N], efeat[:E])


# speedup vs baseline: 3.2810x; 3.2810x over previous
import functools

import jax
import jax.numpy as jnp
from jax import lax
from jax.experimental import pallas as pl
from jax.experimental.pallas import tpu as pltpu
from jax.experimental.pallas import tpu_sc as plsc

N = 10000
E = 5000
D = 128
NNZ = 320000

NC, NS, L = 2, 16, 16
NW = NC * NS

NP = 10240
EP = 5120
C = 128
RT = 80
NNZP = NW * RT * C

PER_T = NNZP // NS
HIST_IT = PER_T // L

_mesh = plsc.VectorSubcoreMesh(core_axis_name="c", subcore_axis_name="s")
_params = pltpu.CompilerParams(needs_layout_passes=False)

_f32 = jnp.float32
_i32 = jnp.int32


def _splat(ref1d, i):
    return plsc.load_gather(ref1d, [jnp.full((L,), i, _i32)])


@functools.partial(
    pl.kernel,
    mesh=_mesh,
    compiler_params=_params,
    out_type=(
        jax.ShapeDtypeStruct((EP,), _f32),
        jax.ShapeDtypeStruct((NP, D), _f32),
    ),
    scratch_types=[
        pltpu.VMEM((PER_T,), _i32),
        pltpu.VMEM((NP,), _f32),
        pltpu.VMEM((EP,), _f32),
        pltpu.VMEM((NP // NS,), _f32),
        pltpu.VMEM((NP // NS,), _f32),
        pltpu.VMEM((320,), _f32),
        pltpu.VMEM((320, D), _f32),
        pltpu.VMEM_SHARED((NS * NP,), _f32),
        pltpu.VMEM_SHARED((NS * EP,), _f32),
        pltpu.VMEM_SHARED((NP,), _f32),
    ],
)
def _stage_a(xpad, nidx, eidx, inv_e_out, xnorm_out,
             idxbuf, histv, histe, abuf, tbuf, sinv, xbuf,
             stagev, stagee, accv):
    c = lax.axis_index("c")
    s = lax.axis_index("s")
    w = s * NC + c
    z16 = jnp.zeros((L,), _f32)
    ones = jnp.ones((L,), _f32)

    def _zv(r, _):
        histv[pl.ds(r * L, L)] = z16
        return 0
    lax.fori_loop(0, NP // L, _zv, 0)

    def _ze(r, _):
        histe[pl.ds(r * L, L)] = z16
        return 0
    lax.fori_loop(0, EP // L, _ze, 0)

    pltpu.sync_copy(nidx.at[pl.ds(s * PER_T, PER_T)], idxbuf)

    def _hv(i, _):
        v = idxbuf[pl.ds(i * L, L)]
        plsc.addupdate_scatter(histv, [v], ones)
        return 0
    lax.fori_loop(0, HIST_IT, _hv, 0)

    pltpu.sync_copy(eidx.at[pl.ds(s * PER_T, PER_T)], idxbuf)

    def _he(i, _):
        v = idxbuf[pl.ds(i * L, L)]
        plsc.addupdate_scatter(histe, [v], ones)
        return 0
    lax.fori_loop(0, HIST_IT, _he, 0)

    pltpu.sync_copy(histv, stagev.at[pl.ds(s * NP, NP)])
    pltpu.sync_copy(histe, stagee.at[pl.ds(s * EP, EP)])
    plsc.subcore_barrier()

    NV = NP // NS
    pltpu.sync_copy(stagev.at[pl.ds(s * NV, NV)], abuf)
    for src in range(1, NS):
        pltpu.sync_copy(stagev.at[pl.ds(src * NP + s * NV, NV)], tbuf)

        def _acc(r, _):
            abuf[pl.ds(r * L, L)] = (abuf[pl.ds(r * L, L)]
                                     + tbuf[pl.ds(r * L, L)])
            return 0
        lax.fori_loop(0, NV // L, _acc, 0)

    def _iv(r, _):
        d = abuf[pl.ds(r * L, L)]
        abuf[pl.ds(r * L, L)] = jnp.where(d > 0.0, 1.0 / d, 0.0)
        return 0
    lax.fori_loop(0, NV // L, _iv, 0)
    pltpu.sync_copy(abuf, accv.at[pl.ds(s * NV, NV)])

    NE = EP // NS
    pltpu.sync_copy(stagee.at[pl.ds(s * NE, NE)], abuf.at[pl.ds(0, NE)])
    for src in range(1, NS):
        pltpu.sync_copy(stagee.at[pl.ds(src * EP + s * NE, NE)],
                        tbuf.at[pl.ds(0, NE)])

        def _acce(r, _):
            abuf[pl.ds(r * L, L)] = (abuf[pl.ds(r * L, L)]
                                     + tbuf[pl.ds(r * L, L)])
            return 0
        lax.fori_loop(0, NE // L, _acce, 0)

    def _ie(r, _):
        d = abuf[pl.ds(r * L, L)]
        abuf[pl.ds(r * L, L)] = jnp.where(d > 0.0, 1.0 / d, 0.0)
        return 0
    lax.fori_loop(0, NE // L, _ie, 0)

    @pl.when(c == 0)
    def _():
        pltpu.sync_copy(abuf.at[pl.ds(0, NE)],
                        inv_e_out.at[pl.ds(s * NE, NE)])
    plsc.subcore_barrier()

    pltpu.sync_copy(xpad.at[pl.ds(w * 320, 320)], xbuf)
    pltpu.sync_copy(accv.at[pl.ds(w * 320, 320)], sinv)

    def _scale(b, _):
        for j in range(L):
            r = b * L + j
            scale = _splat(sinv, r)
            for g in range(D // L):
                xbuf[r, pl.ds(g * L, L)] = xbuf[r, pl.ds(g * L, L)] * scale
        return 0
    lax.fori_loop(0, 20, _scale, 0)
    pltpu.sync_copy(xbuf, xnorm_out.at[pl.ds(w * 320, 320)])


def _make_agg(rows_out):
    @functools.partial(
        pl.kernel,
        mesh=_mesh,
        compiler_params=_params,
        out_type=jax.ShapeDtypeStruct((NC, rows_out, D), _f32),
        scratch_types=[
            pltpu.VMEM((RT, C), _i32),
            pltpu.VMEM((RT, C), _i32),
            pltpu.VMEM((C, D), _f32),
            pltpu.VMEM((64, D), _f32),
            pltpu.VMEM_SHARED((rows_out, D), _f32),
            pltpu.SemaphoreType.DMA,
        ],
    )
    def _agg(table, gidx, sidx, part_out, gbuf, sbuf, rows, zbuf, acc, sem):
        c = lax.axis_index("c")
        s = lax.axis_index("s")
        w = s * NC + c
        per_t = rows_out // NS
        z16 = jnp.zeros((L,), _f32)

        pltpu.sync_copy(gidx.at[pl.ds(w * RT, RT)], gbuf)
        pltpu.sync_copy(sidx.at[pl.ds(w * RT, RT)], sbuf)

        def _z(r, _):
            for g in range(D // L):
                zbuf[r, pl.ds(g * L, L)] = z16
            return 0
        lax.fori_loop(0, 64, _z, 0)
        for k in range(per_t // 64):
            pltpu.sync_copy(zbuf, acc.at[pl.ds(s * per_t + k * 64, 64)])
        plsc.subcore_barrier()

        def _body(j, _):
            pltpu.async_copy(table.at[gbuf.at[j]], rows, sem).wait()
            pltpu.sync_copy(rows, acc.at[sbuf.at[j]], add=True)
            return 0
        lax.fori_loop(0, RT, _body, 0)
        plsc.subcore_barrier()

        for k in range(per_t // 64):
            o = s * per_t + k * 64
            pltpu.sync_copy(acc.at[pl.ds(o, 64)], part_out.at[c, pl.ds(o, 64)])

    return _agg


_v2e = _make_agg(EP)
_e2v = _make_agg(NP)


@functools.partial(
    pl.kernel,
    mesh=_mesh,
    compiler_params=_params,
    out_type=(
        jax.ShapeDtypeStruct((EP, D), _f32),
        jax.ShapeDtypeStruct((EP, D), _f32),
    ),
    scratch_types=[
        pltpu.VMEM((160, D), _f32),
        pltpu.VMEM((160, D), _f32),
        pltpu.VMEM((160, D), _f32),
        pltpu.VMEM((160, D), _f32),
        pltpu.VMEM((160,), _f32),
    ],
)
def _stage_c(part, inv_e, efeat_out, enorm_out, p0, p1, eb, nb, sinv):
    c = lax.axis_index("c")
    s = lax.axis_index("s")
    w = s * NC + c
    pltpu.sync_copy(part.at[0, pl.ds(w * 160, 160)], p0)
    pltpu.sync_copy(part.at[1, pl.ds(w * 160, 160)], p1)
    pltpu.sync_copy(inv_e.at[pl.ds(w * 160, 160)], sinv)

    def _cb(b, _):
        for j in range(L):
            r = b * L + j
            scale = _splat(sinv, r)
            for g in range(D // L):
                v = p0[r, pl.ds(g * L, L)] + p1[r, pl.ds(g * L, L)]
                eb[r, pl.ds(g * L, L)] = v
                nb[r, pl.ds(g * L, L)] = v * scale
        return 0
    lax.fori_loop(0, 10, _cb, 0)
    pltpu.sync_copy(eb, efeat_out.at[pl.ds(w * 160, 160)])
    pltpu.sync_copy(nb, enorm_out.at[pl.ds(w * 160, 160)])


@functools.partial(
    pl.kernel,
    mesh=_mesh,
    compiler_params=_params,
    out_type=jax.ShapeDtypeStruct((NP, D), _f32),
    scratch_types=[
        pltpu.VMEM((160, D), _f32),
        pltpu.VMEM((160, D), _f32),
        pltpu.VMEM((160, D), _f32),
    ],
)
def _stage_e(part, nfeat_out, p0, p1, ob):
    c = lax.axis_index("c")
    s = lax.axis_index("s")
    w = s * NC + c
    for h in range(2):
        base = w * 320 + h * 160
        pltpu.sync_copy(part.at[0, pl.ds(base, 160)], p0)
        pltpu.sync_copy(part.at[1, pl.ds(base, 160)], p1)

        def _ad(r, _):
            for g in range(D // L):
                ob[r, pl.ds(g * L, L)] = (p0[r, pl.ds(g * L, L)]
                                          + p1[r, pl.ds(g * L, L)])
            return 0
        lax.fori_loop(0, 160, _ad, 0)
        pltpu.sync_copy(ob, nfeat_out.at[pl.ds(base, 160)])


def kernel(X, node_idx, edge_idx):
    X = X.astype(_f32)
    nidx = node_idx.astype(_i32)
    eidx = edge_idx.astype(_i32)

    xpad = jnp.zeros((NP, D), _f32).at[:N].set(X)
    nidx_p = jnp.full((NNZP,), NP - 1, _i32).at[:NNZ].set(nidx)
    eidx_p = jnp.full((NNZP,), EP - 1, _i32).at[:NNZ].set(eidx)
    nidx2 = nidx_p.reshape(NNZP // C, C)
    eidx2 = eidx_p.reshape(NNZP // C, C)

    inv_e, xnorm = _stage_a(xpad, nidx_p, eidx_p)
    epart = _v2e(xnorm, nidx2, eidx2)
    efeat, enorm = _stage_c(epart, inv_e)
    npart = _e2v(enorm, eidx2, nidx2)
    nfeat = _stage_e(npart)
    return (nfeat[:N], efeat[:E])
